# baseline (speedup 1.0000x reference)
import functools

import jax
import jax.numpy as jnp
from jax import lax
from jax.experimental import pallas as pl
from jax.experimental.pallas import tpu as pltpu
from jax.experimental.pallas import tpu_sc as plsc

POOLED_H = 7
POOLED_W = 7
SAMPLING = 2
K = SAMPLING * SAMPLING * 4
NUM_CORES = 2
NUM_SUBCORES = 16
LANES = 16
NW = NUM_CORES * NUM_SUBCORES
TBINS = 8


def _prep_body(H, W, scale_ref, rois_ref, idx_ref, w_ref):
    nb = POOLED_H * POOLED_W
    scale = scale_ref[0, 0]
    rois = rois_ref[...]
    b = rois[:, 0:1].astype(jnp.int32)
    x1 = rois[:, 1:2] * scale - 0.5
    y1 = rois[:, 2:3] * scale - 0.5
    x2 = rois[:, 3:4] * scale - 0.5
    y2 = rois[:, 4:5] * scale - 0.5
    bin_w = (x2 - x1) / float(POOLED_W)
    bin_h = (y2 - y1) / float(POOLED_H)
    n = rois.shape[0]

    def sample_coords(bin_i, s_i):
        phf = (bin_i // POOLED_W).astype(jnp.float32)
        pwf = (bin_i % POOLED_W).astype(jnp.float32)
        iyf = (s_i // SAMPLING).astype(jnp.float32)
        ixf = (s_i % SAMPLING).astype(jnp.float32)
        yy = y1 + (phf + (iyf + 0.5) / SAMPLING) * bin_h
        xx = x1 + (pwf + (ixf + 0.5) / SAMPLING) * bin_w
        return yy, xx

    l4 = lax.broadcasted_iota(jnp.int32, (n, nb * 4), 1)
    yy, xx = sample_coords(l4 // 4, l4 % 4)
    y0 = jnp.floor(jnp.clip(yy, 0.0, float(H - 1))).astype(jnp.int32)
    x0 = jnp.floor(jnp.clip(xx, 0.0, float(W - 1))).astype(jnp.int32)
    idx_ref[...] = b * (H * W) + y0 * W + x0

    l16 = lax.broadcasted_iota(jnp.int32, (n, nb * 16), 1)
    r = l16 % 16
    q = r % 4
    yy, xx = sample_coords(l16 // 16, r // 4)
    valid = ((yy > -1.0) & (yy < float(H)) & (xx > -1.0) & (xx < float(W)))
    yc = jnp.clip(yy, 0.0, float(H - 1))
    xc = jnp.clip(xx, 0.0, float(W - 1))
    y0f = jnp.floor(yc)
    x0f = jnp.floor(xc)
    ly = yc - y0f
    lx = xc - x0f
    yclamp = y0f >= float(H - 1)
    xclamp = x0f >= float(W - 1)
    wy = jnp.where((q // 2) == 0,
                   jnp.where(yclamp, 1.0, 1.0 - ly),
                   jnp.where(yclamp, 0.0, ly))
    wx = jnp.where((q % 2) == 0,
                   jnp.where(xclamp, 1.0, 1.0 - lx),
                   jnp.where(xclamp, 0.0, lx))
    w_ref[...] = (wy * wx * valid.astype(jnp.float32)
                  * (1.0 / (SAMPLING * SAMPLING)))


def _cast_transpose_body(x_ref, o_ref):
    o_ref[0] = jnp.transpose(x_ref[0], (1, 2, 0)).astype(jnp.bfloat16)


def _patch_body(W, BHW, C2, ft_hbm, patch_hbm, sem):
    cid = lax.axis_index("core")
    half = BHW // 2

    def copy(src_lo, dst_lo, n, q):
        return pltpu.make_async_copy(
            ft_hbm.at[pl.ds(src_lo, n)],
            patch_hbm.at[pl.ds(dst_lo, n), pl.ds(q * C2, C2)], sem)

    for lo_core in (0, 1):
        @pl.when(cid == lo_core)
        def _():
            copies = []
            for q, shift in enumerate((0, 1, W, W + 1)):
                if lo_core == 0:
                    copies.append(copy(shift, 0, half, q))
                else:
                    copies.append(
                        copy(half + shift, half, BHW - half - shift, q))
                    if shift:
                        copies.append(copy(0, BHW - shift, shift, q))
            for cp in copies:
                cp.start()
            for cp in copies:
                cp.wait()


def _out_body(x_ref, o_ref):
    o_ref[...] = jnp.transpose(x_ref[...], (0, 2, 1)).astype(jnp.float32)


def _sc_body(steps, C, feat_hbm, idx_hbm, w_hbm, out_hbm,
             idx_v, w_v, rows0, rows1, out0, out1,
             gsem0, gsem1, osem0, osem1):
    wid = lax.axis_index("s") * NUM_CORES + lax.axis_index("c")
    base_bin = wid * (TBINS * steps)

    pltpu.sync_copy(idx_hbm.at[pl.ds(base_bin * 4, steps * TBINS * 4)], idx_v)
    pltpu.sync_copy(w_hbm.at[pl.ds(base_bin * K, steps * TBINS * K)], w_v)

    def gather(s, rows, sem):
        return pltpu.make_async_copy(
            feat_hbm.at[idx_v.at[pl.ds(s * TBINS * 4, TBINS * 4)]], rows, sem)

    def outcopy(s, out_v, sem):
        return pltpu.make_async_copy(
            out_v, out_hbm.at[pl.ds(base_bin + s * TBINS, TBINS)], sem)

    def compute(s, rows_v, out_v):
        @plsc.parallel_loop(0, TBINS, 1, unroll=2)
        def _bin(t):
            woff = s * (TBINS * K) + t * K
            wv = [
                plsc.load_gather(
                    w_v, [jnp.full((LANES,), woff + k, dtype=jnp.int32)])
                for k in range(K)
            ]
            r0 = t * 4

            def halves(smp, q, c):
                rv = plsc.bitcast(
                    rows_v[r0 + smp,
                           pl.ds(q * (C // 2) + c * LANES, LANES)],
                    jnp.bfloat16)
                return plsc.unpack(rv, format=plsc.PackFormat.INTERLEAVED)

            for c in range(C // (2 * LANES)):
                acc_e = None
                acc_o = None
                for smp in range(SAMPLING * SAMPLING):
                    for q in range(4):
                        k = smp * 4 + q
                        e, o = halves(smp, q, c)
                        if acc_e is None:
                            acc_e = wv[k] * e
                            acc_o = wv[k] * o
                        else:
                            acc_e = acc_e + wv[k] * e
                            acc_o = acc_o + wv[k] * o
                out_v[t, pl.ds(c * LANES, LANES)] = plsc.bitcast(
                    plsc.pack(acc_e, acc_o,
                              format=plsc.PackFormat.INTERLEAVED),
                    jnp.int32)

    gather(0, rows0, gsem0).start()
    gather(1, rows1, gsem1).start()

    @pl.loop(0, steps // 2)
    def _pair(i):
        s0 = 2 * i
        for par, rows, out_v, gsem, osem in (
                (0, rows0, out0, gsem0, osem0),
                (1, rows1, out1, gsem1, osem1)):
            s = s0 + par
            gather(s, rows, gsem).wait()

            @pl.when(i > 0)
            def _wait_prev_out():
                outcopy(s - 2, out_v, osem).wait()

            compute(s, rows, out_v)
            outcopy(s, out_v, osem).start()

            @pl.when(s + 2 < steps)
            def _next_gather():
                gather(s + 2, rows, gsem).start()

    outcopy(steps - 2, out0, osem0).wait()
    outcopy(steps - 1, out1, osem1).wait()


def kernel(rois, feature, stride):
    N = rois.shape[0]
    B, C, H, W = feature.shape
    nb = POOLED_H * POOLED_W
    bins = N * nb
    steps = -(-bins // (NW * TBINS))
    while (NW * TBINS * steps) % nb or steps % 2:
        steps += 1
    bp = NW * TBINS * steps

    scale = (1.0 / jnp.asarray(stride, dtype=jnp.float32)).reshape(1, 1)
    idx2, w2 = pl.pallas_call(
        functools.partial(_prep_body, H, W),
        out_shape=(
            jax.ShapeDtypeStruct((N, nb * 4), jnp.int32),
            jax.ShapeDtypeStruct((N, nb * K), jnp.float32),
        ),
        in_specs=[
            pl.BlockSpec(memory_space=pltpu.MemorySpace.SMEM),
            pl.BlockSpec(memory_space=pltpu.MemorySpace.VMEM),
        ],
    )(scale, rois[:, :5].astype(jnp.float32))

    idx_flat = jnp.pad(idx2.reshape(bins * 4), (0, (bp - bins) * 4))
    w_flat = jnp.pad(w2.reshape(bins * K), (0, (bp - bins) * K))

    cb = 128 if C % 128 == 0 else C
    ft = pl.pallas_call(
        _cast_transpose_body,
        grid=(B, C // cb),
        in_specs=[pl.BlockSpec((1, cb, H, W), lambda b, c: (b, c, 0, 0))],
        out_specs=pl.BlockSpec((1, H, W, cb), lambda b, c: (b, 0, 0, c)),
        out_shape=jax.ShapeDtypeStruct((B, H, W, C), jnp.bfloat16),
    )(feature).reshape(B * H * W, C)

    ft32 = lax.bitcast_convert_type(
        ft.reshape(B * H * W, C // 2, 2), jnp.int32)
    patch_fn = pl.kernel(
        functools.partial(_patch_body, W, B * H * W, C // 2),
        out_type=jax.ShapeDtypeStruct((B * H * W, 2 * C), jnp.int32),
        mesh=plsc.ScalarSubcoreMesh(axis_name="core", num_cores=NUM_CORES),
        scratch_types=[pltpu.SemaphoreType.DMA],
        compiler_params=pltpu.CompilerParams(use_tc_tiling_on_sc=False),
    )
    featT = patch_fn(ft32)

    sc_fn = pl.kernel(
        functools.partial(_sc_body, steps, C),
        out_type=jax.ShapeDtypeStruct((bp, C // 2), jnp.int32),
        mesh=plsc.VectorSubcoreMesh(
            core_axis_name="c", subcore_axis_name="s",
            num_cores=NUM_CORES, num_subcores=NUM_SUBCORES),
        scratch_types=[
            pltpu.VMEM((steps * TBINS * 4,), jnp.int32),
            pltpu.VMEM((steps * TBINS * K,), jnp.float32),
            pltpu.VMEM((TBINS * 4, 2 * C), jnp.int32),
            pltpu.VMEM((TBINS * 4, 2 * C), jnp.int32),
            pltpu.VMEM((TBINS, C // 2), jnp.int32),
            pltpu.VMEM((TBINS, C // 2), jnp.int32),
            pltpu.SemaphoreType.DMA,
            pltpu.SemaphoreType.DMA,
            pltpu.SemaphoreType.DMA,
            pltpu.SemaphoreType.DMA,
        ],
        compiler_params=pltpu.CompilerParams(needs_layout_passes=False),
    )
    out_flat = sc_fn(featT, idx_flat, w_flat)

    pooled = lax.bitcast_convert_type(
        out_flat, jnp.bfloat16).reshape(bp // nb, nb, C)
    gn = -(-N // 8)
    out = pl.pallas_call(
        _out_body,
        grid=(gn,),
        in_specs=[pl.BlockSpec((8, nb, C), lambda i: (i, 0, 0))],
        out_specs=pl.BlockSpec((8, C, nb), lambda i: (i, 0, 0)),
        out_shape=jax.ShapeDtypeStruct((gn * 8, C, nb), jnp.float32),
    )(pooled)
    return out[:N].reshape(N, C, POOLED_H, POOLED_W)


# speedup vs baseline: 1.4783x; 1.4783x over previous
import functools

import jax
import jax.numpy as jnp
from jax import lax
from jax.experimental import pallas as pl
from jax.experimental.pallas import tpu as pltpu
from jax.experimental.pallas import tpu_sc as plsc

POOLED_H = 7
POOLED_W = 7
SAMPLING = 2
K = SAMPLING * SAMPLING * 4
NUM_CORES = 2
NUM_SUBCORES = 16
LANES = 16
NW = NUM_CORES * NUM_SUBCORES
TBINS = 8


def _prep_body(H, W, scale_ref, rois_ref, idx_ref, w_ref):
    nb = POOLED_H * POOLED_W
    scale = scale_ref[0, 0]
    rois = rois_ref[...]
    b = rois[:, 0:1].astype(jnp.int32)
    x1 = rois[:, 1:2] * scale - 0.5
    y1 = rois[:, 2:3] * scale - 0.5
    x2 = rois[:, 3:4] * scale - 0.5
    y2 = rois[:, 4:5] * scale - 0.5
    bin_w = (x2 - x1) / float(POOLED_W)
    bin_h = (y2 - y1) / float(POOLED_H)
    n = rois.shape[0]

    def sample_coords(bin_i, s_i):
        phf = (bin_i // POOLED_W).astype(jnp.float32)
        pwf = (bin_i % POOLED_W).astype(jnp.float32)
        iyf = (s_i // SAMPLING).astype(jnp.float32)
        ixf = (s_i % SAMPLING).astype(jnp.float32)
        yy = y1 + (phf + (iyf + 0.5) / SAMPLING) * bin_h
        xx = x1 + (pwf + (ixf + 0.5) / SAMPLING) * bin_w
        return yy, xx

    l4 = lax.broadcasted_iota(jnp.int32, (n, nb * 4), 1)
    yy, xx = sample_coords(l4 // 4, l4 % 4)
    y0 = jnp.floor(jnp.clip(yy, 0.0, float(H - 1))).astype(jnp.int32)
    x0 = jnp.floor(jnp.clip(xx, 0.0, float(W - 1))).astype(jnp.int32)
    idx_ref[...] = b * (H * W) + y0 * W + x0

    l16 = lax.broadcasted_iota(jnp.int32, (n, nb * 16), 1)
    r = l16 % 16
    q = r % 4
    yy, xx = sample_coords(l16 // 16, r // 4)
    valid = ((yy > -1.0) & (yy < float(H)) & (xx > -1.0) & (xx < float(W)))
    yc = jnp.clip(yy, 0.0, float(H - 1))
    xc = jnp.clip(xx, 0.0, float(W - 1))
    y0f = jnp.floor(yc)
    x0f = jnp.floor(xc)
    ly = yc - y0f
    lx = xc - x0f
    yclamp = y0f >= float(H - 1)
    xclamp = x0f >= float(W - 1)
    wy = jnp.where((q // 2) == 0,
                   jnp.where(yclamp, 1.0, 1.0 - ly),
                   jnp.where(yclamp, 0.0, ly))
    wx = jnp.where((q % 2) == 0,
                   jnp.where(xclamp, 1.0, 1.0 - lx),
                   jnp.where(xclamp, 0.0, lx))
    w_ref[...] = (wy * wx * valid.astype(jnp.float32)
                  * (1.0 / (SAMPLING * SAMPLING)))


def _cast_transpose_body(x_ref, o_ref):
    o_ref[0] = jnp.transpose(x_ref[0], (1, 2, 0)).astype(jnp.bfloat16)


def _patch_body(BHW, PB, W, ft_ref, out_ref):
    i = pl.program_id(0)
    base = pl.multiple_of(i * PB, 8)
    nxt = pl.multiple_of(jnp.minimum(base + PB, BHW - PB), 8)
    rows_a = ft_ref[pl.ds(base, PB), :]
    rows_b = ft_ref[pl.ds(nxt, PB), :]
    parts = [rows_a]
    for shift in (1, W, W + 1):
        parts.append(jnp.concatenate(
            [rows_a[shift:], rows_b[:shift]], axis=0))
    out_ref[...] = jnp.concatenate(parts, axis=1)


def _out_body(x_ref, o_ref):
    o_ref[...] = jnp.transpose(x_ref[...], (0, 2, 1)).astype(jnp.float32)


def _sc_body(steps, C, feat_hbm, idx_hbm, w_hbm, out_hbm,
             idx_v, w_v, rows0, rows1, out0, out1,
             gsem0, gsem1, osem0, osem1):
    wid = lax.axis_index("s") * NUM_CORES + lax.axis_index("c")
    base_bin = wid * (TBINS * steps)

    pltpu.sync_copy(idx_hbm.at[pl.ds(base_bin * 4, steps * TBINS * 4)], idx_v)
    pltpu.sync_copy(w_hbm.at[pl.ds(base_bin * K, steps * TBINS * K)], w_v)

    def gather(s, rows, sem):
        return pltpu.make_async_copy(
            feat_hbm.at[idx_v.at[pl.ds(s * TBINS * 4, TBINS * 4)]], rows, sem)

    def outcopy(s, out_v, sem):
        return pltpu.make_async_copy(
            out_v, out_hbm.at[pl.ds(base_bin + s * TBINS, TBINS)], sem)

    def compute(s, rows_v, out_v):
        @plsc.parallel_loop(0, TBINS, 1, unroll=2)
        def _bin(t):
            woff = s * (TBINS * K) + t * K
            wv = [
                plsc.load_gather(
                    w_v, [jnp.full((LANES,), woff + k, dtype=jnp.int32)])
                for k in range(K)
            ]
            r0 = t * 4

            def halves(smp, q, c):
                rv = plsc.bitcast(
                    rows_v[r0 + smp,
                           pl.ds(q * (C // 2) + c * LANES, LANES)],
                    jnp.bfloat16)
                return plsc.unpack(rv, format=plsc.PackFormat.INTERLEAVED)

            for c in range(C // (2 * LANES)):
                acc_e = None
                acc_o = None
                for smp in range(SAMPLING * SAMPLING):
                    for q in range(4):
                        k = smp * 4 + q
                        e, o = halves(smp, q, c)
                        if acc_e is None:
                            acc_e = wv[k] * e
                            acc_o = wv[k] * o
                        else:
                            acc_e = acc_e + wv[k] * e
                            acc_o = acc_o + wv[k] * o
                out_v[t, pl.ds(c * LANES, LANES)] = plsc.bitcast(
                    plsc.pack(acc_e, acc_o,
                              format=plsc.PackFormat.INTERLEAVED),
                    jnp.int32)

    gather(0, rows0, gsem0).start()
    gather(1, rows1, gsem1).start()

    @pl.loop(0, steps // 2)
    def _pair(i):
        s0 = 2 * i
        for par, rows, out_v, gsem, osem in (
                (0, rows0, out0, gsem0, osem0),
                (1, rows1, out1, gsem1, osem1)):
            s = s0 + par
            gather(s, rows, gsem).wait()

            @pl.when(i > 0)
            def _wait_prev_out():
                outcopy(s - 2, out_v, osem).wait()

            compute(s, rows, out_v)
            outcopy(s, out_v, osem).start()

            @pl.when(s + 2 < steps)
            def _next_gather():
                gather(s + 2, rows, gsem).start()

    outcopy(steps - 2, out0, osem0).wait()
    outcopy(steps - 1, out1, osem1).wait()


def kernel(rois, feature, stride):
    N = rois.shape[0]
    B, C, H, W = feature.shape
    nb = POOLED_H * POOLED_W
    bins = N * nb
    steps = -(-bins // (NW * TBINS))
    while (NW * TBINS * steps) % nb or steps % 2:
        steps += 1
    bp = NW * TBINS * steps

    scale = (1.0 / jnp.asarray(stride, dtype=jnp.float32)).reshape(1, 1)
    idx2, w2 = pl.pallas_call(
        functools.partial(_prep_body, H, W),
        out_shape=(
            jax.ShapeDtypeStruct((N, nb * 4), jnp.int32),
            jax.ShapeDtypeStruct((N, nb * K), jnp.float32),
        ),
        in_specs=[
            pl.BlockSpec(memory_space=pltpu.MemorySpace.SMEM),
            pl.BlockSpec(memory_space=pltpu.MemorySpace.VMEM),
        ],
    )(scale, rois[:, :5].astype(jnp.float32))

    idx_flat = jnp.pad(idx2.reshape(bins * 4), (0, (bp - bins) * 4))
    w_flat = jnp.pad(w2.reshape(bins * K), (0, (bp - bins) * K))

    cb = 128 if C % 128 == 0 else C
    ft = pl.pallas_call(
        _cast_transpose_body,
        grid=(B, C // cb),
        in_specs=[pl.BlockSpec((1, cb, H, W), lambda b, c: (b, c, 0, 0))],
        out_specs=pl.BlockSpec((1, H, W, cb), lambda b, c: (b, 0, 0, c)),
        out_shape=jax.ShapeDtypeStruct((B, H, W, C), jnp.bfloat16),
    )(feature).reshape(B * H * W, C)

    bhw = B * H * W
    pb = 1000 if bhw % 1000 == 0 else 500
    patch = pl.pallas_call(
        functools.partial(_patch_body, bhw, pb, W),
        grid=(bhw // pb,),
        in_specs=[pl.BlockSpec((bhw, C), lambda i: (0, 0))],
        out_specs=pl.BlockSpec((pb, 4 * C), lambda i: (i, 0)),
        out_shape=jax.ShapeDtypeStruct((bhw, 4 * C), jnp.bfloat16),
    )(ft)
    featT = lax.bitcast_convert_type(
        patch.reshape(bhw, 2 * C, 2), jnp.int32)

    sc_fn = pl.kernel(
        functools.partial(_sc_body, steps, C),
        out_type=jax.ShapeDtypeStruct((bp, C // 2), jnp.int32),
        mesh=plsc.VectorSubcoreMesh(
            core_axis_name="c", subcore_axis_name="s",
            num_cores=NUM_CORES, num_subcores=NUM_SUBCORES),
        scratch_types=[
            pltpu.VMEM((steps * TBINS * 4,), jnp.int32),
            pltpu.VMEM((steps * TBINS * K,), jnp.float32),
            pltpu.VMEM((TBINS * 4, 2 * C), jnp.int32),
            pltpu.VMEM((TBINS * 4, 2 * C), jnp.int32),
            pltpu.VMEM((TBINS, C // 2), jnp.int32),
            pltpu.VMEM((TBINS, C // 2), jnp.int32),
            pltpu.SemaphoreType.DMA,
            pltpu.SemaphoreType.DMA,
            pltpu.SemaphoreType.DMA,
            pltpu.SemaphoreType.DMA,
        ],
        compiler_params=pltpu.CompilerParams(needs_layout_passes=False),
    )
    out_flat = sc_fn(featT, idx_flat, w_flat)

    pooled = lax.bitcast_convert_type(
        out_flat, jnp.bfloat16).reshape(bp // nb, nb, C)
    gn = -(-N // 8)
    out = pl.pallas_call(
        _out_body,
        grid=(gn,),
        in_specs=[pl.BlockSpec((8, nb, C), lambda i: (i, 0, 0))],
        out_specs=pl.BlockSpec((8, C, nb), lambda i: (i, 0, 0)),
        out_shape=jax.ShapeDtypeStruct((gn * 8, C, nb), jnp.float32),
    )(pooled)
    return out[:N].reshape(N, C, POOLED_H, POOLED_W)
